# baseline (speedup 1.0000x reference)
import functools

import jax
import jax.numpy as jnp
import numpy as np
from jax import lax
from jax.experimental import pallas as pl
from jax.experimental.pallas import tpu as pltpu
from jax.experimental.pallas import tpu_sc as plsc

VOCAB = 1000000
EMBED_DIM = 32
SEQ = 50
BATCH = 4096
NC, NS, LANES = 2, 16, 16
NW = NC * NS
BBLK = BATCH // NW
IDXT_PITCH = 136
OB_PITCH = 131
NBUF = 2
NCHUNK = 2 * SEQ


def _body(table_hbm, q_hbm, d_hbm, out_hbm, idxt_v, rows_v, obuf_v, iraw_v,
          gsems, wsems):
    c = lax.axis_index("c")
    s = lax.axis_index("s")
    w = s * NC + c

    iota = lax.iota(jnp.int32, LANES)

    pltpu.sync_copy(q_hbm.at[w], iraw_v.at[0])
    pltpu.sync_copy(d_hbm.at[w], iraw_v.at[1])
    for p in range(2):
        for b in range(BBLK):
            for off in (0, 16, 32, 34):
                v = iraw_v[p, b, pl.ds(off, LANES)]
                dst = (iota + off) * IDXT_PITCH + b
                plsc.store_scatter(idxt_v.at[p], [dst], v)

    for nb in range(NBUF):
        pltpu.async_copy(
            table_hbm.at[idxt_v.at[nb // SEQ, pl.ds((nb % SEQ) * IDXT_PITCH,
                                                    BBLK)]],
            rows_v.at[nb], gsems[nb])

    rt0 = iota // 8
    rt1 = rt0 + 2
    dd0 = iota % 8

    @pl.loop(0, NCHUNK // NBUF)
    def chunk(g):
        for nb in range(NBUF):
            cg = g * NBUF + nb
            p = cg // SEQ
            l = cg - p * SEQ
            pltpu.make_async_copy(
                table_hbm.at[idxt_v.at[p, pl.ds(l * IDXT_PITCH, BBLK)]],
                rows_v.at[nb], gsems[nb]).wait()

            @pl.when(g >= 1)
            def _():
                pg = (cg - NBUF) // SEQ
                lg = (cg - NBUF) - pg * SEQ
                for rt in range(4):
                    pltpu.make_async_copy(
                        obuf_v.at[nb, rt, :, pl.ds(0, BBLK)],
                        out_hbm.at[pg, lg, rt, w], wsems[nb]).wait()

            for bb in range(BBLK):
                v0 = rows_v[nb, bb, pl.ds(0, LANES)]
                v1 = rows_v[nb, bb, pl.ds(LANES, LANES)]
                bbs = jnp.full((LANES,), bb, jnp.int32)
                plsc.store_scatter(obuf_v.at[nb], [rt0, dd0, bbs], v0)
                plsc.store_scatter(obuf_v.at[nb], [rt1, dd0, bbs], v1)
            for rt in range(4):
                pltpu.async_copy(obuf_v.at[nb, rt, :, pl.ds(0, BBLK)],
                                 out_hbm.at[p, l, rt, w], wsems[nb])

            @pl.when(cg + NBUF < NCHUNK)
            def _():
                pn = (cg + NBUF) // SEQ
                ln = (cg + NBUF) - pn * SEQ
                pltpu.async_copy(
                    table_hbm.at[idxt_v.at[pn, pl.ds(ln * IDXT_PITCH, BBLK)]],
                    rows_v.at[nb], gsems[nb])

    for nb in range(NBUF):
        cg = NCHUNK - NBUF + nb
        p = cg // SEQ
        l = cg - p * SEQ
        for rt in range(4):
            pltpu.make_async_copy(obuf_v.at[nb, rt], out_hbm.at[p, l, rt, w],
                                  wsems[nb]).wait()


_TC_WBLK = 512


def _make_sel() -> np.ndarray:
    t = np.arange(_TC_WBLK)[:, None]
    r = np.arange(128)[None, :]
    return np.stack([(t == 4 * r + s) for s in range(4)]).astype(np.float32)


_SEL = _make_sel()


def _tc_relayout_body(tt_ref, sel_ref, out_ref):
    x = tt_ref[...]
    zs = [jax.lax.dot(x, sel_ref[s], precision=jax.lax.Precision.HIGHEST)
          for s in range(4)]
    out_ref[...] = jnp.concatenate(zs, axis=0).T


def _tc_relayout(table_t, sel):
    n = pl.cdiv(VOCAB, _TC_WBLK)
    return pl.pallas_call(
        _tc_relayout_body,
        out_shape=jax.ShapeDtypeStruct((VOCAB // 4, 128), jnp.float32),
        grid=(n,),
        in_specs=[pl.BlockSpec((EMBED_DIM, _TC_WBLK), lambda j: (0, j)),
                  pl.BlockSpec((4, _TC_WBLK, 128), lambda j: (0, 0, 0))],
        out_specs=pl.BlockSpec((_TC_WBLK // 4, 128), lambda j: (j, 0)),
    )(table_t, sel)


@functools.partial(jax.jit, donate_argnums=())
def _run(table, q4, d4):
    mesh = plsc.VectorSubcoreMesh(core_axis_name="c", subcore_axis_name="s",
                                  num_cores=NC, num_subcores=NS)
    kern = pl.kernel(
        _body,
        out_type=jax.ShapeDtypeStruct((2, SEQ, 4, NW, 8, BBLK), jnp.float32),
        mesh=mesh,
        scratch_types=[
            pltpu.VMEM((2, SEQ * IDXT_PITCH), jnp.int32),
            pltpu.VMEM((NBUF, BBLK, EMBED_DIM), jnp.float32),
            pltpu.VMEM((NBUF, 4, 8, OB_PITCH), jnp.float32),
            pltpu.VMEM((2, BBLK, SEQ), jnp.int32),
            [pltpu.SemaphoreType.DMA] * NBUF,
            [pltpu.SemaphoreType.DMA] * NBUF,
        ],
        compiler_params=pltpu.CompilerParams(needs_layout_passes=False,
                                             use_tc_tiling_on_sc=False),
    )
    table_lin = _tc_relayout(table.T, jnp.asarray(_SEL)).reshape(VOCAB,
                                                                 EMBED_DIM)
    return kern(table_lin, q4, d4)


def kernel(query_input, document_input, table):
    q4 = query_input.astype(jnp.int32).reshape(NW, BBLK, SEQ)
    d4 = document_input.astype(jnp.int32).reshape(NW, BBLK, SEQ)
    out6 = _run(table, q4, d4)
    return out6.transpose(0, 3, 5, 2, 4, 1).reshape(2, BATCH, EMBED_DIM, SEQ)


# speedup vs baseline: 1.2409x; 1.2409x over previous
import functools

import jax
import jax.numpy as jnp
import numpy as np
from jax import lax
from jax.experimental import pallas as pl
from jax.experimental.pallas import tpu as pltpu
from jax.experimental.pallas import tpu_sc as plsc

VOCAB = 1000000
EMBED_DIM = 32
SEQ = 50
BATCH = 4096
NC, NS, LANES = 2, 16, 16
NW = NC * NS
BBLK = BATCH // NW
IDXT_PITCH = 136
OB_PITCH = 131
NBUF = 2
NCHUNK = 2 * SEQ


def _body(table_hbm, q_hbm, d_hbm, out_hbm, idxt_v, rows_v, obuf_v, iraw_v,
          gsems, wsems):
    c = lax.axis_index("c")
    s = lax.axis_index("s")
    w = s * NC + c

    iota = lax.iota(jnp.int32, LANES)

    pltpu.sync_copy(q_hbm.at[w], iraw_v.at[0])
    pltpu.sync_copy(d_hbm.at[w], iraw_v.at[1])
    for p in range(2):
        for b in range(BBLK):
            for off in (0, 16, 32, 34):
                v = iraw_v[p, b, pl.ds(off, LANES)]
                dst = (iota + off) * IDXT_PITCH + b
                plsc.store_scatter(idxt_v.at[p], [dst], v)

    for nb in range(NBUF):
        pltpu.async_copy(
            table_hbm.at[idxt_v.at[nb // SEQ, pl.ds((nb % SEQ) * IDXT_PITCH,
                                                    BBLK)]],
            rows_v.at[nb], gsems[nb])

    rt0 = iota // 8
    rt1 = rt0 + 2
    dd0 = iota % 8

    @pl.loop(0, NCHUNK // NBUF)
    def chunk(g):
        for nb in range(NBUF):
            cg = g * NBUF + nb
            p = cg // SEQ
            l = cg - p * SEQ
            pltpu.make_async_copy(
                table_hbm.at[idxt_v.at[p, pl.ds(l * IDXT_PITCH, BBLK)]],
                rows_v.at[nb], gsems[nb]).wait()

            @pl.when(g >= 1)
            def _():
                pg = (cg - NBUF) // SEQ
                lg = (cg - NBUF) - pg * SEQ
                for rt in range(4):
                    pltpu.make_async_copy(
                        obuf_v.at[nb, rt, :, pl.ds(0, BBLK)],
                        out_hbm.at[pg, lg, rt, w], wsems[nb]).wait()

            for bb in range(BBLK):
                v0 = rows_v[nb, bb, pl.ds(0, LANES)]
                v1 = rows_v[nb, bb, pl.ds(LANES, LANES)]
                bbs = jnp.full((LANES,), bb, jnp.int32)
                plsc.store_scatter(obuf_v.at[nb], [rt0, dd0, bbs], v0)
                plsc.store_scatter(obuf_v.at[nb], [rt1, dd0, bbs], v1)
            for rt in range(4):
                pltpu.async_copy(obuf_v.at[nb, rt, :, pl.ds(0, BBLK)],
                                 out_hbm.at[p, l, rt, w], wsems[nb])

            @pl.when(cg + NBUF < NCHUNK)
            def _():
                pn = (cg + NBUF) // SEQ
                ln = (cg + NBUF) - pn * SEQ
                pltpu.async_copy(
                    table_hbm.at[idxt_v.at[pn, pl.ds(ln * IDXT_PITCH, BBLK)]],
                    rows_v.at[nb], gsems[nb])

    for nb in range(NBUF):
        cg = NCHUNK - NBUF + nb
        p = cg // SEQ
        l = cg - p * SEQ
        for rt in range(4):
            pltpu.make_async_copy(obuf_v.at[nb, rt], out_hbm.at[p, l, rt, w],
                                  wsems[nb]).wait()


_TC_WBLK = 512


def _make_sel() -> np.ndarray:
    t = np.arange(_TC_WBLK)[:, None]
    r = np.arange(128)[None, :]
    return np.stack([(t == 4 * r + s) for s in range(4)]).astype(np.float32)


_SEL = _make_sel()


def _tc_relayout_body(tt_ref, sel_ref, out_ref):
    x = tt_ref[...]
    zs = [jax.lax.dot(x, sel_ref[s], precision=jax.lax.Precision.DEFAULT)
          for s in range(4)]
    out_ref[...] = jnp.concatenate(zs, axis=0).T


def _tc_relayout(table_t, sel):
    n = pl.cdiv(VOCAB, _TC_WBLK)
    return pl.pallas_call(
        _tc_relayout_body,
        out_shape=jax.ShapeDtypeStruct((VOCAB // 4, 128), jnp.float32),
        grid=(n,),
        in_specs=[pl.BlockSpec((EMBED_DIM, _TC_WBLK), lambda j: (0, j)),
                  pl.BlockSpec((4, _TC_WBLK, 128), lambda j: (0, 0, 0))],
        out_specs=pl.BlockSpec((_TC_WBLK // 4, 128), lambda j: (j, 0)),
    )(table_t, sel)


@functools.partial(jax.jit, donate_argnums=())
def _run(table, q4, d4):
    mesh = plsc.VectorSubcoreMesh(core_axis_name="c", subcore_axis_name="s",
                                  num_cores=NC, num_subcores=NS)
    kern = pl.kernel(
        _body,
        out_type=jax.ShapeDtypeStruct((2, SEQ, 4, NW, 8, BBLK), jnp.float32),
        mesh=mesh,
        scratch_types=[
            pltpu.VMEM((2, SEQ * IDXT_PITCH), jnp.int32),
            pltpu.VMEM((NBUF, BBLK, EMBED_DIM), jnp.float32),
            pltpu.VMEM((NBUF, 4, 8, OB_PITCH), jnp.float32),
            pltpu.VMEM((2, BBLK, SEQ), jnp.int32),
            [pltpu.SemaphoreType.DMA] * NBUF,
            [pltpu.SemaphoreType.DMA] * NBUF,
        ],
        compiler_params=pltpu.CompilerParams(needs_layout_passes=False,
                                             use_tc_tiling_on_sc=False),
    )
    table_lin = _tc_relayout(table.T, jnp.asarray(_SEL)).reshape(VOCAB,
                                                                 EMBED_DIM)
    return kern(table_lin, q4, d4)


def kernel(query_input, document_input, table):
    q4 = query_input.astype(jnp.int32).reshape(NW, BBLK, SEQ)
    d4 = document_input.astype(jnp.int32).reshape(NW, BBLK, SEQ)
    out6 = _run(table, q4, d4)
    return out6.transpose(0, 3, 5, 2, 4, 1).reshape(2, BATCH, EMBED_DIM, SEQ)


# speedup vs baseline: 2.6775x; 2.1577x over previous
import functools

import jax
import jax.numpy as jnp
import numpy as np
from jax import lax
from jax.experimental import pallas as pl
from jax.experimental.pallas import tpu as pltpu
from jax.experimental.pallas import tpu_sc as plsc

VOCAB = 1000000
EMBED_DIM = 32
SEQ = 50
BATCH = 4096
NC, NS, LANES = 2, 16, 16
NW = NC * NS
BBLK = BATCH // NW
IDXT_PITCH = 136
OB_PITCH = 131
NBUF = 2
NCHUNK = 2 * SEQ


def _body(table_hbm, q_hbm, d_hbm, out_hbm, idxt_v, rows_v, obuf_v, iraw_v,
          gsems, wsems):
    c = lax.axis_index("c")
    s = lax.axis_index("s")
    w = s * NC + c

    iota = lax.iota(jnp.int32, LANES)

    pltpu.sync_copy(q_hbm.at[w], iraw_v.at[0])
    pltpu.sync_copy(d_hbm.at[w], iraw_v.at[1])
    for p in range(2):
        for b in range(BBLK):
            for off in (0, 16, 32, 34):
                v = iraw_v[p, b, pl.ds(off, LANES)]
                dst = (iota + off) * IDXT_PITCH + b
                plsc.store_scatter(idxt_v.at[p], [dst], v)

    for nb in range(NBUF):
        pltpu.async_copy(
            table_hbm.at[idxt_v.at[nb // SEQ, pl.ds((nb % SEQ) * IDXT_PITCH,
                                                    BBLK)]],
            rows_v.at[nb], gsems[nb])

    rt0 = iota // 8
    rt1 = rt0 + 2
    dd0 = iota % 8

    @pl.loop(0, NCHUNK // NBUF)
    def chunk(g):
        for nb in range(NBUF):
            cg = g * NBUF + nb
            p = cg // SEQ
            l = cg - p * SEQ
            pltpu.make_async_copy(
                table_hbm.at[idxt_v.at[p, pl.ds(l * IDXT_PITCH, BBLK)]],
                rows_v.at[nb], gsems[nb]).wait()

            @pl.when(g >= 1)
            def _():
                pg = (cg - NBUF) // SEQ
                lg = (cg - NBUF) - pg * SEQ
                for rt in range(4):
                    pltpu.make_async_copy(
                        obuf_v.at[nb, rt, :, pl.ds(0, BBLK)],
                        out_hbm.at[pg, lg, rt, w], wsems[nb]).wait()

            for bb in range(BBLK):
                v0 = rows_v[nb, bb, pl.ds(0, LANES)]
                v1 = rows_v[nb, bb, pl.ds(LANES, LANES)]
                bbs = jnp.full((LANES,), bb, jnp.int32)
                plsc.store_scatter(obuf_v.at[nb], [rt0, dd0, bbs], v0)
                plsc.store_scatter(obuf_v.at[nb], [rt1, dd0, bbs], v1)
            for rt in range(4):
                pltpu.async_copy(obuf_v.at[nb, rt, :, pl.ds(0, BBLK)],
                                 out_hbm.at[p, l, rt, w], wsems[nb])

            @pl.when(cg + NBUF < NCHUNK)
            def _():
                pn = (cg + NBUF) // SEQ
                ln = (cg + NBUF) - pn * SEQ
                pltpu.async_copy(
                    table_hbm.at[idxt_v.at[pn, pl.ds(ln * IDXT_PITCH, BBLK)]],
                    rows_v.at[nb], gsems[nb])

    for nb in range(NBUF):
        cg = NCHUNK - NBUF + nb
        p = cg // SEQ
        l = cg - p * SEQ
        for rt in range(4):
            pltpu.make_async_copy(obuf_v.at[nb, rt, :, pl.ds(0, BBLK)],
                                  out_hbm.at[p, l, rt, w], wsems[nb]).wait()


@functools.partial(jax.jit, donate_argnums=())
def _run(table, q4, d4):
    mesh = plsc.VectorSubcoreMesh(core_axis_name="c", subcore_axis_name="s",
                                  num_cores=NC, num_subcores=NS)
    kern = pl.kernel(
        _body,
        out_type=jax.ShapeDtypeStruct((2, SEQ, 4, NW, 8, BBLK), jnp.float32),
        mesh=mesh,
        scratch_types=[
            pltpu.VMEM((2, SEQ * IDXT_PITCH), jnp.int32),
            pltpu.VMEM((NBUF, BBLK, EMBED_DIM), jnp.float32),
            pltpu.VMEM((NBUF, 4, 8, OB_PITCH), jnp.float32),
            pltpu.VMEM((2, BBLK, SEQ), jnp.int32),
            [pltpu.SemaphoreType.DMA] * NBUF,
            [pltpu.SemaphoreType.DMA] * NBUF,
        ],
        compiler_params=pltpu.CompilerParams(needs_layout_passes=False,
                                             use_tc_tiling_on_sc=False),
    )
    return kern(table, q4, d4)


def kernel(query_input, document_input, table):
    q4 = query_input.astype(jnp.int32).reshape(NW, BBLK, SEQ)
    d4 = document_input.astype(jnp.int32).reshape(NW, BBLK, SEQ)
    out6 = _run(table, q4, d4)
    return out6.transpose(0, 3, 5, 2, 4, 1).reshape(2, BATCH, EMBED_DIM, SEQ)
